# baseline (speedup 1.0000x reference)
import functools
import jax
import jax.numpy as jnp
from jax import lax
from jax.experimental import pallas as pl
from jax.experimental.pallas import tpu as pltpu
from jax.experimental.pallas import tpu_sc as plsc

DQ = 16
NQ = 4
EG = 128
GPC = 8
CH = EG * GPC
NTILES = 16


def _seg_sum_body(nchunks, rpn, src_hbm, dst_hbm, h0, h1, h2, h3, z_hbm,
                  t0, t1, t2, t3, sidx, didx, rows, acc,
                  gsem0, gsem1, ssem0, ssem1):
    c = lax.axis_index("c")
    s = lax.axis_index("s")
    gsem = (gsem0, gsem1)
    ssem = (ssem0, ssem1)

    def do_quarter(h_hbm, t_hbm):
        pltpu.sync_copy(z_hbm, acc.at[pl.ds(s * rpn, rpn)])
        plsc.subcore_barrier()

        def issue(i, b):
            row0 = (s * nchunks + i) * GPC
            pltpu.sync_copy(src_hbm.at[pl.ds(row0, GPC)], sidx.at[b])
            pltpu.sync_copy(dst_hbm.at[pl.ds(row0, GPC)], didx.at[b])
            for j in range(GPC):
                pltpu.async_copy(h_hbm.at[sidx.at[b].at[j]],
                                 rows.at[b].at[pl.ds(j * EG, EG)], gsem[b])

        def wait_gathers(b):
            for j in range(GPC):
                pltpu.make_async_copy(
                    h_hbm.at[sidx.at[b].at[j]],
                    rows.at[b].at[pl.ds(j * EG, EG)], gsem[b]).wait()

        def fire_scatters(b):
            for j in range(GPC):
                pltpu.async_copy(rows.at[b].at[pl.ds(j * EG, EG)],
                                 acc.at[didx.at[b].at[j]], ssem[b], add=True)

        def drain_scatters(b):
            for j in range(GPC):
                pltpu.make_async_copy(
                    rows.at[b].at[pl.ds(j * EG, EG)],
                    acc.at[didx.at[b].at[j]], ssem[b]).wait()

        issue(0, 0)
        issue(1, 1)

        def body(g, _):
            for b in range(2):
                i = 2 * g + b
                wait_gathers(b)
                fire_scatters(b)
                drain_scatters(b)
                issue(i + 2, b)
            return 0

        lax.fori_loop(0, nchunks // 2 - 1, body, 0)
        for b in range(2):
            wait_gathers(b)
            fire_scatters(b)
            drain_scatters(b)
        plsc.subcore_barrier()
        pltpu.sync_copy(acc.at[pl.ds(s * rpn, rpn)],
                        t_hbm.at[pl.ds(s * rpn, rpn)])

    @pl.when(c == 0)
    def _():
        do_quarter(h0, t0)
        do_quarter(h1, t1)

    @pl.when(c == 1)
    def _():
        do_quarter(h2, t2)
        do_quarter(h3, t3)


def _deg_body(nchunks, rpn, dst_hbm, onesq_hbm, z_hbm, ta, tb,
              didx, rows, acc, ssem0, ssem1):
    c = lax.axis_index("c")
    s = lax.axis_index("s")
    ssem = (ssem0, ssem1)

    pltpu.sync_copy(onesq_hbm.at[pl.ds(0, CH)], rows)
    pltpu.sync_copy(z_hbm, acc.at[pl.ds(s * rpn, rpn)])
    plsc.subcore_barrier()

    base_chunk = (c * NTILES + s) * nchunks

    def load_idx(i, b):
        pltpu.sync_copy(dst_hbm.at[pl.ds((base_chunk + i) * GPC, GPC)],
                        didx.at[b])

    def fire_scatters(b):
        for j in range(GPC):
            pltpu.async_copy(rows.at[pl.ds(j * EG, EG)],
                             acc.at[didx.at[b].at[j]], ssem[b], add=True)

    def drain_scatters(b):
        for j in range(GPC):
            pltpu.make_async_copy(rows.at[pl.ds(j * EG, EG)],
                                  acc.at[didx.at[b].at[j]], ssem[b]).wait()

    load_idx(0, 0)
    fire_scatters(0)
    load_idx(1, 1)
    fire_scatters(1)

    def body(g, _):
        for b in range(2):
            i = 2 * g + b
            drain_scatters(b)
            load_idx(i + 2, b)
            fire_scatters(b)
        return 0

    lax.fori_loop(0, nchunks // 2 - 1, body, 0)
    for b in range(2):
        drain_scatters(b)
    plsc.subcore_barrier()

    @pl.when(c == 0)
    def _():
        pltpu.sync_copy(acc.at[pl.ds(s * rpn, rpn)],
                        ta.at[pl.ds(s * rpn, rpn)])

    @pl.when(c == 1)
    def _():
        pltpu.sync_copy(acc.at[pl.ds(s * rpn, rpn)],
                        tb.at[pl.ds(s * rpn, rpn)])


def _make_deg(npad, ne_pad):
    nchunks = ne_pad // (2 * NTILES * CH)
    rpn = npad // NTILES
    mesh = plsc.VectorSubcoreMesh(
        core_axis_name="c", subcore_axis_name="s",
        num_cores=2, num_subcores=NTILES)
    return functools.partial(
        pl.kernel,
        mesh=mesh,
        compiler_params=pltpu.CompilerParams(use_tc_tiling_on_sc=False),
        out_type=(jax.ShapeDtypeStruct((npad, DQ), jnp.float32),
                  jax.ShapeDtypeStruct((npad, DQ), jnp.float32)),
        scratch_types=[
            pltpu.VMEM((2, GPC, EG), jnp.int32),
            pltpu.VMEM((CH, DQ), jnp.float32),
            pltpu.VMEM_SHARED((npad, DQ), jnp.float32),
            pltpu.SemaphoreType.DMA,
            pltpu.SemaphoreType.DMA,
        ],
    )(functools.partial(_deg_body, nchunks, rpn))


def _make_seg_sum(npad, ne_pad):
    nchunks = ne_pad // (NTILES * CH)
    rpn = npad // NTILES
    mesh = plsc.VectorSubcoreMesh(
        core_axis_name="c", subcore_axis_name="s",
        num_cores=2, num_subcores=NTILES)
    return functools.partial(
        pl.kernel,
        mesh=mesh,
        compiler_params=pltpu.CompilerParams(use_tc_tiling_on_sc=False),
        out_type=tuple(jax.ShapeDtypeStruct((npad, DQ), jnp.float32)
                       for _ in range(NQ)),
        scratch_types=[
            pltpu.VMEM((2, GPC, EG), jnp.int32),
            pltpu.VMEM((2, GPC, EG), jnp.int32),
            pltpu.VMEM((2, CH, DQ), jnp.float32),
            pltpu.VMEM_SHARED((npad, DQ), jnp.float32),
            pltpu.SemaphoreType.DMA,
            pltpu.SemaphoreType.DMA,
            pltpu.SemaphoreType.DMA,
            pltpu.SemaphoreType.DMA,
        ],
    )(functools.partial(_seg_sum_body, nchunks, rpn))


def _prep_tc(emb_blk, ta_blk, tb_blk, dinv_blk, h0_blk, h1_blk, h2_blk,
             h3_blk):
    deg = ta_blk[:, 0:1] + tb_blk[:, 0:1]
    dinv = jnp.where(deg > 0.0,
                     jax.lax.rsqrt(jnp.maximum(deg, 1e-12)), 0.0)
    dinvq = jnp.broadcast_to(dinv, (deg.shape[0], DQ))
    dinv_blk[...] = dinvq
    for q, h_blk in enumerate((h0_blk, h1_blk, h2_blk, h3_blk)):
        h_blk[...] = dinvq * emb_blk[:, q * DQ:(q + 1) * DQ]


def _finalize_tc(t0_blk, t1_blk, t2_blk, t3_blk, dinv_blk, tot_blk,
                 h0_blk, h1_blk, h2_blk, h3_blk, out_blk):
    dinvq = dinv_blk[...]
    hs = (h0_blk, h1_blk, h2_blk, h3_blk)
    for q, t_blk in enumerate((t0_blk, t1_blk, t2_blk, t3_blk)):
        g = dinvq * t_blk[...]
        hs[q][...] = dinvq * g
        out_blk[:, q * DQ:(q + 1) * DQ] = tot_blk[:, q * DQ:(q + 1) * DQ] + g


def _last_tc(t0_blk, t1_blk, t2_blk, t3_blk, dinv_blk, tot_blk, out_blk):
    dinvq = dinv_blk[...]
    for q, t_blk in enumerate((t0_blk, t1_blk, t2_blk, t3_blk)):
        out_blk[:, q * DQ:(q + 1) * DQ] = (
            tot_blk[:, q * DQ:(q + 1) * DQ] + dinvq * t_blk[...]) * 0.25


def kernel(user_emb, item_emb, user_idx, item_idx):
    n_users = user_emb.shape[0]
    n_items = item_emb.shape[0]
    n_nodes = n_users + n_items
    n_edges = user_idx.shape[0]
    dim = user_emb.shape[1]

    npad = ((n_nodes + 1 + 127) // 128) * 128
    ne = 2 * n_edges
    qt = 4 * NTILES * CH
    ne_pad = ((ne + qt - 1) // qt) * qt
    rpn = npad // NTILES

    src = jnp.concatenate([user_idx, item_idx + n_users])
    dst = jnp.concatenate([item_idx + n_users, user_idx])
    pad = jnp.full((ne_pad - ne,), n_nodes, dtype=jnp.int32)
    src2 = jnp.concatenate([src, pad]).reshape(ne_pad // EG, EG)
    dst2 = jnp.concatenate([dst, pad]).reshape(ne_pad // EG, EG)

    emb = jnp.concatenate([user_emb, item_emb], axis=0)
    emb = jnp.concatenate(
        [emb, jnp.zeros((npad - n_nodes, dim), jnp.float32)], axis=0)
    onesq = jnp.ones((npad, DQ), jnp.float32)
    z = jnp.zeros((rpn, DQ), jnp.float32)

    seg_sum = _make_seg_sum(npad, ne_pad)

    ta, tb = _make_deg(npad, ne_pad)(dst2, onesq, z)

    nblk = 16
    rb = npad // nblk
    row_specq = pl.BlockSpec((rb, DQ), lambda i: (i, 0))
    row_spec64 = pl.BlockSpec((rb, dim), lambda i: (i, 0))
    sdq = jax.ShapeDtypeStruct((npad, DQ), jnp.float32)
    dinvq, h0, h1, h2, h3 = pl.pallas_call(
        _prep_tc,
        grid=(nblk,),
        in_specs=[row_spec64, row_specq, row_specq],
        out_specs=[row_specq] * 5,
        out_shape=[sdq] * 5,
    )(emb, ta, tb)

    total = emb
    for layer in range(3):
        t0, t1, t2, t3 = seg_sum(src2, dst2, h0, h1, h2, h3, z)
        if layer < 2:
            h0, h1, h2, h3, total = pl.pallas_call(
                _finalize_tc,
                grid=(nblk,),
                in_specs=[row_specq] * 5 + [row_spec64],
                out_specs=[row_specq] * 4 + [row_spec64],
                out_shape=[sdq] * 4 +
                          [jax.ShapeDtypeStruct((npad, dim), jnp.float32)],
            )(t0, t1, t2, t3, dinvq, total)
        else:
            out = pl.pallas_call(
                _last_tc,
                grid=(nblk,),
                in_specs=[row_specq] * 5 + [row_spec64],
                out_specs=row_spec64,
                out_shape=jax.ShapeDtypeStruct((npad, dim), jnp.float32),
            )(t0, t1, t2, t3, dinvq, total)

    return out[:n_users], out[n_users:n_nodes]


# speedup vs baseline: 1.5997x; 1.5997x over previous
import functools
import jax
import jax.numpy as jnp
from jax import lax
from jax.experimental import pallas as pl
from jax.experimental.pallas import tpu as pltpu
from jax.experimental.pallas import tpu_sc as plsc

DQ = 16
NQ = 4
EG = 128
GPC = 8
CH = EG * GPC
NTILES = 16


def _seg_sum_body(nchunks, rpn, src_hbm, dst_hbm, h0, h1, h2, h3, z_hbm,
                  t0, t1, t2, t3, sidx, didx, rows, acc,
                  gsem0, gsem1, ssem0, ssem1):
    c = lax.axis_index("c")
    s = lax.axis_index("s")
    gsem = (gsem0, gsem1)
    ssem = (ssem0, ssem1)

    def do_quarter(h_hbm, t_hbm):
        pltpu.sync_copy(z_hbm, acc.at[pl.ds(s * rpn, rpn)])
        plsc.subcore_barrier()

        def issue(i, b):
            row0 = (s * nchunks + i) * GPC
            pltpu.sync_copy(src_hbm.at[pl.ds(row0, GPC)], sidx.at[b])
            pltpu.sync_copy(dst_hbm.at[pl.ds(row0, GPC)], didx.at[b])
            for j in range(GPC):
                pltpu.async_copy(h_hbm.at[sidx.at[b].at[j]],
                                 rows.at[b].at[pl.ds(j * EG, EG)], gsem[b])

        def wait_gathers(b):
            for j in range(GPC):
                pltpu.make_async_copy(
                    h_hbm.at[sidx.at[b].at[j]],
                    rows.at[b].at[pl.ds(j * EG, EG)], gsem[b]).wait()

        def fire_scatters(b):
            for j in range(GPC):
                pltpu.async_copy(rows.at[b].at[pl.ds(j * EG, EG)],
                                 acc.at[didx.at[b].at[j]], ssem[b], add=True)

        def drain_scatters(b):
            for j in range(GPC):
                pltpu.make_async_copy(
                    rows.at[b].at[pl.ds(j * EG, EG)],
                    acc.at[didx.at[b].at[j]], ssem[b]).wait()

        issue(0, 0)
        issue(1, 1)

        def body(g, _):
            for b in range(2):
                i = 2 * g + b
                wait_gathers(b)
                fire_scatters(b)
                drain_scatters(b)
                issue(i + 2, b)
            return 0

        lax.fori_loop(0, nchunks // 2 - 1, body, 0)
        for b in range(2):
            wait_gathers(b)
            fire_scatters(b)
            drain_scatters(b)
        plsc.subcore_barrier()
        pltpu.sync_copy(acc.at[pl.ds(s * rpn, rpn)],
                        t_hbm.at[pl.ds(s * rpn, rpn)])

    @pl.when(c == 0)
    def _():
        do_quarter(h0, t0)
        do_quarter(h1, t1)

    @pl.when(c == 1)
    def _():
        do_quarter(h2, t2)
        do_quarter(h3, t3)


def _deg_body(nchunks, rpn, dst_hbm, onesq_hbm, z_hbm, ta, tb,
              didx, rows, acc, ssem0, ssem1):
    c = lax.axis_index("c")
    s = lax.axis_index("s")
    ssem = (ssem0, ssem1)

    pltpu.sync_copy(onesq_hbm.at[pl.ds(0, CH)], rows)
    pltpu.sync_copy(z_hbm, acc.at[pl.ds(s * rpn, rpn)])
    plsc.subcore_barrier()

    base_chunk = (c * NTILES + s) * nchunks

    def load_idx(i, b):
        pltpu.sync_copy(dst_hbm.at[pl.ds((base_chunk + i) * GPC, GPC)],
                        didx.at[b])

    def fire_scatters(b):
        for j in range(GPC):
            pltpu.async_copy(rows.at[pl.ds(j * EG, EG)],
                             acc.at[didx.at[b].at[j]], ssem[b], add=True)

    def drain_scatters(b):
        for j in range(GPC):
            pltpu.make_async_copy(rows.at[pl.ds(j * EG, EG)],
                                  acc.at[didx.at[b].at[j]], ssem[b]).wait()

    load_idx(0, 0)
    fire_scatters(0)
    load_idx(1, 1)
    fire_scatters(1)

    def body(g, _):
        for b in range(2):
            i = 2 * g + b
            drain_scatters(b)
            load_idx(i + 2, b)
            fire_scatters(b)
        return 0

    lax.fori_loop(0, nchunks // 2 - 1, body, 0)
    for b in range(2):
        drain_scatters(b)
    plsc.subcore_barrier()

    @pl.when(c == 0)
    def _():
        pltpu.sync_copy(acc.at[pl.ds(s * rpn, rpn)],
                        ta.at[pl.ds(s * rpn, rpn)])

    @pl.when(c == 1)
    def _():
        pltpu.sync_copy(acc.at[pl.ds(s * rpn, rpn)],
                        tb.at[pl.ds(s * rpn, rpn)])


def _make_deg(npad, ne_pad):
    nchunks = ne_pad // (2 * NTILES * CH)
    rpn = npad // NTILES
    mesh = plsc.VectorSubcoreMesh(
        core_axis_name="c", subcore_axis_name="s",
        num_cores=2, num_subcores=NTILES)
    return functools.partial(
        pl.kernel,
        mesh=mesh,
        compiler_params=pltpu.CompilerParams(use_tc_tiling_on_sc=False),
        out_type=(jax.ShapeDtypeStruct((npad, DQ), jnp.float32),
                  jax.ShapeDtypeStruct((npad, DQ), jnp.float32)),
        scratch_types=[
            pltpu.VMEM((2, GPC, EG), jnp.int32),
            pltpu.VMEM((CH, DQ), jnp.float32),
            pltpu.VMEM_SHARED((npad, DQ), jnp.float32),
            pltpu.SemaphoreType.DMA,
            pltpu.SemaphoreType.DMA,
        ],
    )(functools.partial(_deg_body, nchunks, rpn))


def _make_seg_sum(npad, ne_pad):
    nchunks = ne_pad // (NTILES * CH)
    rpn = npad // NTILES
    mesh = plsc.VectorSubcoreMesh(
        core_axis_name="c", subcore_axis_name="s",
        num_cores=2, num_subcores=NTILES)
    return functools.partial(
        pl.kernel,
        mesh=mesh,
        compiler_params=pltpu.CompilerParams(use_tc_tiling_on_sc=False),
        out_type=tuple(jax.ShapeDtypeStruct((npad, DQ), jnp.float32)
                       for _ in range(NQ)),
        scratch_types=[
            pltpu.VMEM((2, GPC, EG), jnp.int32),
            pltpu.VMEM((2, GPC, EG), jnp.int32),
            pltpu.VMEM((2, CH, DQ), jnp.float32),
            pltpu.VMEM_SHARED((npad, DQ), jnp.float32),
            pltpu.SemaphoreType.DMA,
            pltpu.SemaphoreType.DMA,
            pltpu.SemaphoreType.DMA,
            pltpu.SemaphoreType.DMA,
        ],
    )(functools.partial(_seg_sum_body, nchunks, rpn))


def _prep_tc(emb_blk, ta_blk, tb_blk, dinv_blk, h0_blk, h1_blk, h2_blk,
             h3_blk):
    deg = ta_blk[:, 0:1] + tb_blk[:, 0:1]
    dinv = jnp.where(deg > 0.0,
                     jax.lax.rsqrt(jnp.maximum(deg, 1e-12)), 0.0)
    dinvq = jnp.broadcast_to(dinv, (deg.shape[0], DQ))
    dinv_blk[...] = dinvq
    for q, h_blk in enumerate((h0_blk, h1_blk, h2_blk, h3_blk)):
        h_blk[...] = dinvq * emb_blk[:, q * DQ:(q + 1) * DQ]


def _finalize_tc(t0_blk, t1_blk, t2_blk, t3_blk, dinv_blk, tot_blk,
                 h0_blk, h1_blk, h2_blk, h3_blk, out_blk):
    dinvq = dinv_blk[...]
    hs = (h0_blk, h1_blk, h2_blk, h3_blk)
    for q, t_blk in enumerate((t0_blk, t1_blk, t2_blk, t3_blk)):
        g = dinvq * t_blk[...]
        hs[q][...] = dinvq * g
        out_blk[:, q * DQ:(q + 1) * DQ] = tot_blk[:, q * DQ:(q + 1) * DQ] + g


def _last_tc(t0_blk, t1_blk, t2_blk, t3_blk, dinv_blk, tot_blk, out_blk):
    dinvq = dinv_blk[...]
    for q, t_blk in enumerate((t0_blk, t1_blk, t2_blk, t3_blk)):
        out_blk[:, q * DQ:(q + 1) * DQ] = (
            tot_blk[:, q * DQ:(q + 1) * DQ] + dinvq * t_blk[...]) * 0.25


def kernel(user_emb, item_emb, user_idx, item_idx):
    n_users = user_emb.shape[0]
    n_items = item_emb.shape[0]
    n_nodes = n_users + n_items
    n_edges = user_idx.shape[0]
    dim = user_emb.shape[1]

    ndummy = 1024
    npad = ((n_nodes + ndummy + 127) // 128) * 128
    ne = 2 * n_edges
    qt = 4 * NTILES * CH
    ne_pad = ((ne + qt - 1) // qt) * qt
    rpn = npad // NTILES

    src = jnp.concatenate([user_idx, item_idx + n_users])
    dst = jnp.concatenate([item_idx + n_users, user_idx])
    pad = n_nodes + (jnp.arange(ne_pad - ne, dtype=jnp.int32) % ndummy)
    src2 = jnp.concatenate([src, pad]).reshape(ne_pad // EG, EG)
    dst2 = jnp.concatenate([dst, pad]).reshape(ne_pad // EG, EG)

    emb = jnp.concatenate([user_emb, item_emb], axis=0)
    emb = jnp.concatenate(
        [emb, jnp.zeros((npad - n_nodes, dim), jnp.float32)], axis=0)
    onesq = jnp.ones((npad, DQ), jnp.float32)
    z = jnp.zeros((rpn, DQ), jnp.float32)

    seg_sum = _make_seg_sum(npad, ne_pad)

    ta, tb = _make_deg(npad, ne_pad)(dst2, onesq, z)

    nblk = 16
    rb = npad // nblk
    row_specq = pl.BlockSpec((rb, DQ), lambda i: (i, 0))
    row_spec64 = pl.BlockSpec((rb, dim), lambda i: (i, 0))
    sdq = jax.ShapeDtypeStruct((npad, DQ), jnp.float32)
    dinvq, h0, h1, h2, h3 = pl.pallas_call(
        _prep_tc,
        grid=(nblk,),
        in_specs=[row_spec64, row_specq, row_specq],
        out_specs=[row_specq] * 5,
        out_shape=[sdq] * 5,
    )(emb, ta, tb)

    total = emb
    for layer in range(3):
        t0, t1, t2, t3 = seg_sum(src2, dst2, h0, h1, h2, h3, z)
        if layer < 2:
            h0, h1, h2, h3, total = pl.pallas_call(
                _finalize_tc,
                grid=(nblk,),
                in_specs=[row_specq] * 5 + [row_spec64],
                out_specs=[row_specq] * 4 + [row_spec64],
                out_shape=[sdq] * 4 +
                          [jax.ShapeDtypeStruct((npad, dim), jnp.float32)],
            )(t0, t1, t2, t3, dinvq, total)
        else:
            out = pl.pallas_call(
                _last_tc,
                grid=(nblk,),
                in_specs=[row_specq] * 5 + [row_spec64],
                out_specs=row_spec64,
                out_shape=jax.ShapeDtypeStruct((npad, dim), jnp.float32),
            )(t0, t1, t2, t3, dinvq, total)

    return out[:n_users], out[n_users:n_nodes]


# speedup vs baseline: 1.7382x; 1.0866x over previous
import functools
import jax
import jax.numpy as jnp
from jax import lax
from jax.experimental import pallas as pl
from jax.experimental.pallas import tpu as pltpu
from jax.experimental.pallas import tpu_sc as plsc

DQ = 16
DH = 32
EG = 128
GPC = 3
CH = EG * GPC
NTILES = 16


def _seg_sum_body(nchunks, rpn, src_hbm, dst_hbm, h0, h1, z_hbm,
                  t0, t1, sidx, didx, rows, acc,
                  gsem0, gsem1, ssem0, ssem1):
    c = lax.axis_index("c")
    s = lax.axis_index("s")
    gsem = (gsem0, gsem1)
    ssem = (ssem0, ssem1)

    def do_half(h_hbm, t_hbm):
        pltpu.sync_copy(z_hbm, acc.at[pl.ds(s * rpn, rpn)])
        plsc.subcore_barrier()

        def issue(i, b):
            row0 = (s * nchunks + i) * GPC
            pltpu.sync_copy(src_hbm.at[pl.ds(row0, GPC)], sidx.at[b])
            pltpu.sync_copy(dst_hbm.at[pl.ds(row0, GPC)], didx.at[b])
            for j in range(GPC):
                pltpu.async_copy(h_hbm.at[sidx.at[b].at[j]],
                                 rows.at[b].at[pl.ds(j * EG, EG)], gsem[b])

        def wait_gathers(b):
            for j in range(GPC):
                pltpu.make_async_copy(
                    h_hbm.at[sidx.at[b].at[j]],
                    rows.at[b].at[pl.ds(j * EG, EG)], gsem[b]).wait()

        def fire_scatters(b):
            for j in range(GPC):
                pltpu.async_copy(rows.at[b].at[pl.ds(j * EG, EG)],
                                 acc.at[didx.at[b].at[j]], ssem[b], add=True)

        def drain_scatters(b):
            for j in range(GPC):
                pltpu.make_async_copy(
                    rows.at[b].at[pl.ds(j * EG, EG)],
                    acc.at[didx.at[b].at[j]], ssem[b]).wait()

        issue(0, 0)
        issue(1, 1)

        def body(g, _):
            for b in range(2):
                i = 2 * g + b
                wait_gathers(b)
                fire_scatters(b)
                drain_scatters(b)
                issue(i + 2, b)
            return 0

        lax.fori_loop(0, nchunks // 2 - 1, body, 0)
        for b in range(2):
            wait_gathers(b)
            fire_scatters(b)
            drain_scatters(b)
        plsc.subcore_barrier()
        pltpu.sync_copy(acc.at[pl.ds(s * rpn, rpn)],
                        t_hbm.at[pl.ds(s * rpn, rpn)])

    @pl.when(c == 0)
    def _():
        do_half(h0, t0)

    @pl.when(c == 1)
    def _():
        do_half(h1, t1)


def _deg_body(nchunks, rpn, dst_hbm, onesq_hbm, z_hbm, ta, tb,
              didx, rows, acc, ssem0, ssem1):
    c = lax.axis_index("c")
    s = lax.axis_index("s")
    ssem = (ssem0, ssem1)

    pltpu.sync_copy(onesq_hbm.at[pl.ds(0, CH)], rows)
    pltpu.sync_copy(z_hbm, acc.at[pl.ds(s * rpn, rpn)])
    plsc.subcore_barrier()

    base_chunk = (c * NTILES + s) * nchunks

    def load_idx(i, b):
        pltpu.sync_copy(dst_hbm.at[pl.ds((base_chunk + i) * GPC, GPC)],
                        didx.at[b])

    def fire_scatters(b):
        for j in range(GPC):
            pltpu.async_copy(rows.at[pl.ds(j * EG, EG)],
                             acc.at[didx.at[b].at[j]], ssem[b], add=True)

    def drain_scatters(b):
        for j in range(GPC):
            pltpu.make_async_copy(rows.at[pl.ds(j * EG, EG)],
                                  acc.at[didx.at[b].at[j]], ssem[b]).wait()

    load_idx(0, 0)
    fire_scatters(0)
    load_idx(1, 1)
    fire_scatters(1)

    def body(g, _):
        for b in range(2):
            i = 2 * g + b
            drain_scatters(b)
            load_idx(i + 2, b)
            fire_scatters(b)
        return 0

    lax.fori_loop(0, nchunks // 2 - 1, body, 0)
    for b in range(2):
        drain_scatters(b)
    plsc.subcore_barrier()

    @pl.when(c == 0)
    def _():
        pltpu.sync_copy(acc.at[pl.ds(s * rpn, rpn)],
                        ta.at[pl.ds(s * rpn, rpn)])

    @pl.when(c == 1)
    def _():
        pltpu.sync_copy(acc.at[pl.ds(s * rpn, rpn)],
                        tb.at[pl.ds(s * rpn, rpn)])


def _make_deg(npad, ne_pad):
    nchunks = ne_pad // (2 * NTILES * CH)
    rpn = npad // NTILES
    mesh = plsc.VectorSubcoreMesh(
        core_axis_name="c", subcore_axis_name="s",
        num_cores=2, num_subcores=NTILES)
    return functools.partial(
        pl.kernel,
        mesh=mesh,
        compiler_params=pltpu.CompilerParams(use_tc_tiling_on_sc=False),
        out_type=(jax.ShapeDtypeStruct((npad, DQ), jnp.float32),
                  jax.ShapeDtypeStruct((npad, DQ), jnp.float32)),
        scratch_types=[
            pltpu.VMEM((2, GPC, EG), jnp.int32),
            pltpu.VMEM((CH, DQ), jnp.float32),
            pltpu.VMEM_SHARED((npad, DQ), jnp.float32),
            pltpu.SemaphoreType.DMA,
            pltpu.SemaphoreType.DMA,
        ],
    )(functools.partial(_deg_body, nchunks, rpn))


def _make_seg_sum(npad, ne_pad):
    nchunks = ne_pad // (NTILES * CH)
    rpn = npad // NTILES
    mesh = plsc.VectorSubcoreMesh(
        core_axis_name="c", subcore_axis_name="s",
        num_cores=2, num_subcores=NTILES)
    return functools.partial(
        pl.kernel,
        mesh=mesh,
        compiler_params=pltpu.CompilerParams(use_tc_tiling_on_sc=False),
        out_type=(jax.ShapeDtypeStruct((npad, DH), jnp.float32),
                  jax.ShapeDtypeStruct((npad, DH), jnp.float32)),
        scratch_types=[
            pltpu.VMEM((2, GPC, EG), jnp.int32),
            pltpu.VMEM((2, GPC, EG), jnp.int32),
            pltpu.VMEM((2, CH, DH), jnp.float32),
            pltpu.VMEM_SHARED((npad, DH), jnp.float32),
            pltpu.SemaphoreType.DMA,
            pltpu.SemaphoreType.DMA,
            pltpu.SemaphoreType.DMA,
            pltpu.SemaphoreType.DMA,
        ],
    )(functools.partial(_seg_sum_body, nchunks, rpn))


def _prep_tc(emb_blk, ta_blk, tb_blk, dinv_blk, h0_blk, h1_blk):
    deg = ta_blk[:, 0:1] + tb_blk[:, 0:1]
    dinv = jnp.where(deg > 0.0,
                     jax.lax.rsqrt(jnp.maximum(deg, 1e-12)), 0.0)
    dinvh = jnp.broadcast_to(dinv, (deg.shape[0], DH))
    dinv_blk[...] = dinvh
    h0_blk[...] = dinvh * emb_blk[:, :DH]
    h1_blk[...] = dinvh * emb_blk[:, DH:]


def _finalize_tc(t0_blk, t1_blk, dinv_blk, tot_blk, h0_blk, h1_blk, out_blk):
    dinvh = dinv_blk[...]
    g0 = dinvh * t0_blk[...]
    g1 = dinvh * t1_blk[...]
    h0_blk[...] = dinvh * g0
    h1_blk[...] = dinvh * g1
    out_blk[:, :DH] = tot_blk[:, :DH] + g0
    out_blk[:, DH:] = tot_blk[:, DH:] + g1


def _last_tc(t0_blk, t1_blk, dinv_blk, tot_blk, out_blk):
    dinvh = dinv_blk[...]
    out_blk[:, :DH] = (tot_blk[:, :DH] + dinvh * t0_blk[...]) * 0.25
    out_blk[:, DH:] = (tot_blk[:, DH:] + dinvh * t1_blk[...]) * 0.25


def kernel(user_emb, item_emb, user_idx, item_idx):
    n_users = user_emb.shape[0]
    n_items = item_emb.shape[0]
    n_nodes = n_users + n_items
    n_edges = user_idx.shape[0]
    dim = user_emb.shape[1]

    ndummy = 512
    npad = ((n_nodes + ndummy + 127) // 128) * 128
    ne = 2 * n_edges
    qt = 4 * NTILES * CH
    ne_pad = ((ne + qt - 1) // qt) * qt
    rpn = npad // NTILES

    src = jnp.concatenate([user_idx, item_idx + n_users])
    dst = jnp.concatenate([item_idx + n_users, user_idx])
    pad = n_nodes + (jnp.arange(ne_pad - ne, dtype=jnp.int32) % ndummy)
    src2 = jnp.concatenate([src, pad]).reshape(ne_pad // EG, EG)
    dst2 = jnp.concatenate([dst, pad]).reshape(ne_pad // EG, EG)

    emb = jnp.concatenate([user_emb, item_emb], axis=0)
    emb = jnp.concatenate(
        [emb, jnp.zeros((npad - n_nodes, dim), jnp.float32)], axis=0)
    onesq = jnp.ones((npad, DQ), jnp.float32)
    z16 = jnp.zeros((rpn, DQ), jnp.float32)
    z32 = jnp.zeros((rpn, DH), jnp.float32)

    seg_sum = _make_seg_sum(npad, ne_pad)

    ta, tb = _make_deg(npad, ne_pad)(dst2, onesq, z16)

    nblk = 16
    rb = npad // nblk
    row_spec16 = pl.BlockSpec((rb, DQ), lambda i: (i, 0))
    row_spec32 = pl.BlockSpec((rb, DH), lambda i: (i, 0))
    row_spec64 = pl.BlockSpec((rb, dim), lambda i: (i, 0))
    sdh = jax.ShapeDtypeStruct((npad, DH), jnp.float32)
    dinvh, h0, h1 = pl.pallas_call(
        _prep_tc,
        grid=(nblk,),
        in_specs=[row_spec64, row_spec16, row_spec16],
        out_specs=[row_spec32] * 3,
        out_shape=[sdh] * 3,
    )(emb, ta, tb)

    total = emb
    for layer in range(3):
        t0, t1 = seg_sum(src2, dst2, h0, h1, z32)
        if layer < 2:
            h0, h1, total = pl.pallas_call(
                _finalize_tc,
                grid=(nblk,),
                in_specs=[row_spec32] * 3 + [row_spec64],
                out_specs=[row_spec32, row_spec32, row_spec64],
                out_shape=[sdh, sdh,
                           jax.ShapeDtypeStruct((npad, dim), jnp.float32)],
            )(t0, t1, dinvh, total)
        else:
            out = pl.pallas_call(
                _last_tc,
                grid=(nblk,),
                in_specs=[row_spec32] * 3 + [row_spec64],
                out_specs=row_spec64,
                out_shape=jax.ShapeDtypeStruct((npad, dim), jnp.float32),
            )(t0, t1, dinvh, total)

    return out[:n_users], out[n_users:n_nodes]


# speedup vs baseline: 2.1976x; 1.2643x over previous
import functools
import math
import jax
import jax.numpy as jnp
from jax import lax
from jax.experimental import pallas as pl
from jax.experimental.pallas import tpu as pltpu
from jax.experimental.pallas import tpu_sc as plsc

DQ = 16
DH = 32
EG = 128
GPC = 2
CH = EG * GPC
NTILES = 16
NBUF = 3


def _seg_sum_body(nchunks, rpn, comb_hbm, h0, h1, z_hbm,
                  t0, t1, cidx, rows, acc,
                  isem0, isem1, isem2, gsem0, gsem1, gsem2,
                  ssem0, ssem1, ssem2):
    c = lax.axis_index("c")
    s = lax.axis_index("s")
    isem = (isem0, isem1, isem2)
    gsem = (gsem0, gsem1, gsem2)
    ssem = (ssem0, ssem1, ssem2)

    def do_half(h_hbm, t_hbm):
        pltpu.sync_copy(z_hbm, acc.at[pl.ds(s * rpn, rpn)])
        plsc.subcore_barrier()

        def idx_copy(i, b):
            row0 = (s * nchunks + i) * 2 * GPC
            return pltpu.make_async_copy(
                comb_hbm.at[pl.ds(row0, 2 * GPC)], cidx.at[b], isem[b])

        def gather_copy(b, j):
            return pltpu.make_async_copy(
                h_hbm.at[cidx.at[b].at[j]],
                rows.at[b].at[pl.ds(j * EG, EG)], gsem[b])

        def idx_start(i, b):
            idx_copy(i, b).start()

        def fire_gathers(b):
            for j in range(GPC):
                gather_copy(b, j).start()

        def wait_idx(i, b):
            idx_copy(i, b).wait()

        def wait_gathers(b):
            for j in range(GPC):
                gather_copy(b, j).wait()

        def fire_scatters(b):
            for j in range(GPC):
                pltpu.async_copy(rows.at[b].at[pl.ds(j * EG, EG)],
                                 acc.at[cidx.at[b].at[GPC + j]],
                                 ssem[b], add=True)

        def drain_scatters(b):
            for j in range(GPC):
                pltpu.make_async_copy(
                    rows.at[b].at[pl.ds(j * EG, EG)],
                    acc.at[cidx.at[b].at[GPC + j]], ssem[b]).wait()

        def complete(cc, b, b2, refill):
            wait_gathers(b)
            fire_scatters(b)
            drain_scatters(b)
            if refill:
                idx_start(cc + NBUF, b)
                wait_idx(cc + 2, b2)
                fire_gathers(b2)

        for b in range(NBUF):
            idx_start(b, b)
        for b in range(2):
            wait_idx(b, b)
            fire_gathers(b)

        def body(g, _):
            for m in range(NBUF):
                cc = NBUF * g + m
                complete(cc, m, (m + 2) % NBUF, True)
            return 0

        lax.fori_loop(0, (nchunks - NBUF) // NBUF, body, 0)
        base = nchunks - NBUF
        for m in range(NBUF):
            cc = base + m
            b = cc % NBUF
            wait_gathers(b)
            fire_scatters(b)
            drain_scatters(b)
            if m == 0:
                wait_idx(cc + 2, (cc + 2) % NBUF)
                fire_gathers((cc + 2) % NBUF)
        plsc.subcore_barrier()
        pltpu.sync_copy(acc.at[pl.ds(s * rpn, rpn)],
                        t_hbm.at[pl.ds(s * rpn, rpn)])

    @pl.when(c == 0)
    def _():
        do_half(h0, t0)

    @pl.when(c == 1)
    def _():
        do_half(h1, t1)


def _deg_body(nchunks, rpn, dst_hbm, onesq_hbm, z_hbm, ta, tb,
              didx, rows, acc, ssem0, ssem1):
    c = lax.axis_index("c")
    s = lax.axis_index("s")
    ssem = (ssem0, ssem1)

    pltpu.sync_copy(onesq_hbm.at[pl.ds(0, CH)], rows)
    pltpu.sync_copy(z_hbm, acc.at[pl.ds(s * rpn, rpn)])
    plsc.subcore_barrier()

    base_chunk = (c * NTILES + s) * nchunks

    def load_idx(i, b):
        pltpu.sync_copy(dst_hbm.at[pl.ds((base_chunk + i) * GPC, GPC)],
                        didx.at[b])

    def fire_scatters(b):
        for j in range(GPC):
            pltpu.async_copy(rows.at[pl.ds(j * EG, EG)],
                             acc.at[didx.at[b].at[j]], ssem[b], add=True)

    def drain_scatters(b):
        for j in range(GPC):
            pltpu.make_async_copy(rows.at[pl.ds(j * EG, EG)],
                                  acc.at[didx.at[b].at[j]], ssem[b]).wait()

    load_idx(0, 0)
    fire_scatters(0)
    load_idx(1, 1)
    fire_scatters(1)

    def body(g, _):
        for b in range(2):
            i = 2 * g + b
            drain_scatters(b)
            load_idx(i + 2, b)
            fire_scatters(b)
        return 0

    lax.fori_loop(0, nchunks // 2 - 1, body, 0)
    for b in range(2):
        drain_scatters(b)
    plsc.subcore_barrier()

    @pl.when(c == 0)
    def _():
        pltpu.sync_copy(acc.at[pl.ds(s * rpn, rpn)],
                        ta.at[pl.ds(s * rpn, rpn)])

    @pl.when(c == 1)
    def _():
        pltpu.sync_copy(acc.at[pl.ds(s * rpn, rpn)],
                        tb.at[pl.ds(s * rpn, rpn)])


def _make_deg(npad, ne_pad):
    nchunks = ne_pad // (2 * NTILES * CH)
    rpn = npad // NTILES
    mesh = plsc.VectorSubcoreMesh(
        core_axis_name="c", subcore_axis_name="s",
        num_cores=2, num_subcores=NTILES)
    return functools.partial(
        pl.kernel,
        mesh=mesh,
        compiler_params=pltpu.CompilerParams(use_tc_tiling_on_sc=False),
        out_type=(jax.ShapeDtypeStruct((npad, DQ), jnp.float32),
                  jax.ShapeDtypeStruct((npad, DQ), jnp.float32)),
        scratch_types=[
            pltpu.VMEM((2, GPC, EG), jnp.int32),
            pltpu.VMEM((CH, DQ), jnp.float32),
            pltpu.VMEM_SHARED((npad, DQ), jnp.float32),
            pltpu.SemaphoreType.DMA,
            pltpu.SemaphoreType.DMA,
        ],
    )(functools.partial(_deg_body, nchunks, rpn))


def _make_seg_sum(npad, ne_pad):
    nchunks = ne_pad // (NTILES * CH)
    rpn = npad // NTILES
    mesh = plsc.VectorSubcoreMesh(
        core_axis_name="c", subcore_axis_name="s",
        num_cores=2, num_subcores=NTILES)
    return functools.partial(
        pl.kernel,
        mesh=mesh,
        compiler_params=pltpu.CompilerParams(use_tc_tiling_on_sc=False),
        out_type=(jax.ShapeDtypeStruct((npad, DH), jnp.float32),
                  jax.ShapeDtypeStruct((npad, DH), jnp.float32)),
        scratch_types=[
            pltpu.VMEM((NBUF, 2 * GPC, EG), jnp.int32),
            pltpu.VMEM((NBUF, CH, DH), jnp.float32),
            pltpu.VMEM_SHARED((npad, DH), jnp.float32),
        ] + [pltpu.SemaphoreType.DMA] * 9,
    )(functools.partial(_seg_sum_body, nchunks, rpn))


def _prep_tc(emb_blk, ta_blk, tb_blk, dinv_blk, h0_blk, h1_blk):
    deg = ta_blk[:, 0:1] + tb_blk[:, 0:1]
    dinv = jnp.where(deg > 0.0,
                     jax.lax.rsqrt(jnp.maximum(deg, 1e-12)), 0.0)
    dinvh = jnp.broadcast_to(dinv, (deg.shape[0], DH))
    dinv_blk[...] = dinvh
    h0_blk[...] = dinvh * emb_blk[:, :DH]
    h1_blk[...] = dinvh * emb_blk[:, DH:]


def _finalize_tc(t0_blk, t1_blk, dinv_blk, tot_blk, h0_blk, h1_blk, out_blk):
    dinvh = dinv_blk[...]
    g0 = dinvh * t0_blk[...]
    g1 = dinvh * t1_blk[...]
    h0_blk[...] = dinvh * g0
    h1_blk[...] = dinvh * g1
    out_blk[:, :DH] = tot_blk[:, :DH] + g0
    out_blk[:, DH:] = tot_blk[:, DH:] + g1


def _last_tc(t0_blk, t1_blk, dinv_blk, tot_blk, out_blk):
    dinvh = dinv_blk[...]
    out_blk[:, :DH] = (tot_blk[:, :DH] + dinvh * t0_blk[...]) * 0.25
    out_blk[:, DH:] = (tot_blk[:, DH:] + dinvh * t1_blk[...]) * 0.25


def kernel(user_emb, item_emb, user_idx, item_idx):
    n_users = user_emb.shape[0]
    n_items = item_emb.shape[0]
    n_nodes = n_users + n_items
    n_edges = user_idx.shape[0]
    dim = user_emb.shape[1]

    ndummy = 512
    npad = ((n_nodes + ndummy + 127) // 128) * 128
    ne = 2 * n_edges
    qt = NTILES * CH * NBUF
    qd = 4 * NTILES * CH
    lcm = qt * qd // math.gcd(qt, qd)
    ne_pad = ((ne + lcm - 1) // lcm) * lcm
    rpn = npad // NTILES

    src = jnp.concatenate([user_idx, item_idx + n_users])
    dst = jnp.concatenate([item_idx + n_users, user_idx])
    pad = n_nodes + (jnp.arange(ne_pad - ne, dtype=jnp.int32) % ndummy)
    srcp = jnp.concatenate([src, pad])
    dstp = jnp.concatenate([dst, pad])
    dst2 = dstp.reshape(ne_pad // EG, EG)
    tchunks = ne_pad // CH
    comb = jnp.concatenate(
        [srcp.reshape(tchunks, GPC, EG), dstp.reshape(tchunks, GPC, EG)],
        axis=1).reshape(tchunks * 2 * GPC, EG)

    emb = jnp.concatenate([user_emb, item_emb], axis=0)
    emb = jnp.concatenate(
        [emb, jnp.zeros((npad - n_nodes, dim), jnp.float32)], axis=0)
    onesq = jnp.ones((npad, DQ), jnp.float32)
    z16 = jnp.zeros((rpn, DQ), jnp.float32)
    z32 = jnp.zeros((rpn, DH), jnp.float32)

    seg_sum = _make_seg_sum(npad, ne_pad)

    ta, tb = _make_deg(npad, ne_pad)(dst2, onesq, z16)

    nblk = 16
    rb = npad // nblk
    row_spec16 = pl.BlockSpec((rb, DQ), lambda i: (i, 0))
    row_spec32 = pl.BlockSpec((rb, DH), lambda i: (i, 0))
    row_spec64 = pl.BlockSpec((rb, dim), lambda i: (i, 0))
    sdh = jax.ShapeDtypeStruct((npad, DH), jnp.float32)
    dinvh, h0, h1 = pl.pallas_call(
        _prep_tc,
        grid=(nblk,),
        in_specs=[row_spec64, row_spec16, row_spec16],
        out_specs=[row_spec32] * 3,
        out_shape=[sdh] * 3,
    )(emb, ta, tb)

    total = emb
    for layer in range(3):
        t0, t1 = seg_sum(comb, h0, h1, z32)
        if layer < 2:
            h0, h1, total = pl.pallas_call(
                _finalize_tc,
                grid=(nblk,),
                in_specs=[row_spec32] * 3 + [row_spec64],
                out_specs=[row_spec32, row_spec32, row_spec64],
                out_shape=[sdh, sdh,
                           jax.ShapeDtypeStruct((npad, dim), jnp.float32)],
            )(t0, t1, dinvh, total)
        else:
            out = pl.pallas_call(
                _last_tc,
                grid=(nblk,),
                in_specs=[row_spec32] * 3 + [row_spec64],
                out_specs=row_spec64,
                out_shape=jax.ShapeDtypeStruct((npad, dim), jnp.float32),
            )(t0, t1, dinvh, total)

    return out[:n_users], out[n_users:n_nodes]


# speedup vs baseline: 2.2431x; 1.0207x over previous
import functools
import math
import jax
import jax.numpy as jnp
from jax import lax
from jax.experimental import pallas as pl
from jax.experimental.pallas import tpu as pltpu
from jax.experimental.pallas import tpu_sc as plsc

DQ = 16
DH = 32
EG = 128
GPC = 2
CH = EG * GPC
NTILES = 16
NBUF = 3


def _seg_sum_body(nchunks, rpn, comb_hbm, h0, h1, z_hbm,
                  t0, t1, cidx, rows, acc,
                  isem0, isem1, isem2, gsem0, gsem1, gsem2,
                  ssem0, ssem1, ssem2):
    c = lax.axis_index("c")
    s = lax.axis_index("s")
    isem = (isem0, isem1, isem2)
    gsem = (gsem0, gsem1, gsem2)
    ssem = (ssem0, ssem1, ssem2)

    def do_half(h_hbm, t_hbm):
        pltpu.sync_copy(z_hbm, acc.at[pl.ds(s * rpn, rpn)])
        plsc.subcore_barrier()

        def idx_copy(i, b):
            row0 = (s * nchunks + i) * 2 * GPC
            return pltpu.make_async_copy(
                comb_hbm.at[pl.ds(row0, 2 * GPC)], cidx.at[b], isem[b])

        def gather_copy(b, j):
            return pltpu.make_async_copy(
                h_hbm.at[cidx.at[b].at[j]],
                rows.at[b].at[pl.ds(j * EG, EG)], gsem[b])

        def idx_start(i, b):
            idx_copy(i, b).start()

        def fire_gathers(b):
            for j in range(GPC):
                gather_copy(b, j).start()

        def wait_idx(i, b):
            idx_copy(i, b).wait()

        def wait_gathers(b):
            for j in range(GPC):
                gather_copy(b, j).wait()

        def fire_scatters(b):
            for j in range(GPC):
                pltpu.async_copy(rows.at[b].at[pl.ds(j * EG, EG)],
                                 acc.at[cidx.at[b].at[GPC + j]],
                                 ssem[b], add=True)

        def drain_scatters(b):
            for j in range(GPC):
                pltpu.make_async_copy(
                    rows.at[b].at[pl.ds(j * EG, EG)],
                    acc.at[cidx.at[b].at[GPC + j]], ssem[b]).wait()

        def complete(cc, b, b2, refill):
            wait_gathers(b)
            fire_scatters(b)
            drain_scatters(b)
            if refill:
                idx_start(cc + NBUF, b)
                wait_idx(cc + 2, b2)
                fire_gathers(b2)

        for b in range(NBUF):
            idx_start(b, b)
        for b in range(2):
            wait_idx(b, b)
            fire_gathers(b)

        def body(g, _):
            for m in range(NBUF):
                cc = NBUF * g + m
                complete(cc, m, (m + 2) % NBUF, True)
            return 0

        lax.fori_loop(0, (nchunks - NBUF) // NBUF, body, 0)
        base = nchunks - NBUF
        for m in range(NBUF):
            cc = base + m
            b = cc % NBUF
            wait_gathers(b)
            fire_scatters(b)
            drain_scatters(b)
            if m == 0:
                wait_idx(cc + 2, (cc + 2) % NBUF)
                fire_gathers((cc + 2) % NBUF)
        plsc.subcore_barrier()
        pltpu.sync_copy(acc.at[pl.ds(s * rpn, rpn)],
                        t_hbm.at[pl.ds(s * rpn, rpn)])

    @pl.when(c == 0)
    def _():
        do_half(h0, t0)

    @pl.when(c == 1)
    def _():
        do_half(h1, t1)


def _deg_body(nchunks, rpn, dst_hbm, onesq_hbm, z_hbm, ta, tb,
              didx, rows, acc, isem0, isem1, ssem0, ssem1):
    c = lax.axis_index("c")
    s = lax.axis_index("s")
    isem = (isem0, isem1)
    ssem = (ssem0, ssem1)

    pltpu.sync_copy(onesq_hbm.at[pl.ds(0, CH)], rows)
    pltpu.sync_copy(z_hbm, acc.at[pl.ds(s * rpn, rpn)])
    plsc.subcore_barrier()

    base_chunk = (c * NTILES + s) * nchunks

    def idx_copy(i, b):
        return pltpu.make_async_copy(
            dst_hbm.at[pl.ds((base_chunk + i) * GPC, GPC)],
            didx.at[b], isem[b])

    def fire_scatters(b):
        for j in range(GPC):
            pltpu.async_copy(rows.at[pl.ds(j * EG, EG)],
                             acc.at[didx.at[b].at[j]], ssem[b], add=True)

    def drain_scatters(b):
        for j in range(GPC):
            pltpu.make_async_copy(rows.at[pl.ds(j * EG, EG)],
                                  acc.at[didx.at[b].at[j]], ssem[b]).wait()

    idx_copy(0, 0).start()
    idx_copy(1, 1).start()

    def body(g, _):
        for b in range(2):
            i = 2 * g + b
            idx_copy(i, b).wait()
            fire_scatters(b)
            drain_scatters(b)
            idx_copy(i + 2, b).start()
        return 0

    lax.fori_loop(0, nchunks // 2 - 1, body, 0)
    base = nchunks - 2
    for b in range(2):
        idx_copy(base + b, b).wait()
        fire_scatters(b)
        drain_scatters(b)
    plsc.subcore_barrier()

    @pl.when(c == 0)
    def _():
        pltpu.sync_copy(acc.at[pl.ds(s * rpn, rpn)],
                        ta.at[pl.ds(s * rpn, rpn)])

    @pl.when(c == 1)
    def _():
        pltpu.sync_copy(acc.at[pl.ds(s * rpn, rpn)],
                        tb.at[pl.ds(s * rpn, rpn)])


def _make_deg(npad, ne_pad):
    nchunks = ne_pad // (2 * NTILES * CH)
    rpn = npad // NTILES
    mesh = plsc.VectorSubcoreMesh(
        core_axis_name="c", subcore_axis_name="s",
        num_cores=2, num_subcores=NTILES)
    return functools.partial(
        pl.kernel,
        mesh=mesh,
        compiler_params=pltpu.CompilerParams(use_tc_tiling_on_sc=False),
        out_type=(jax.ShapeDtypeStruct((npad, DQ), jnp.float32),
                  jax.ShapeDtypeStruct((npad, DQ), jnp.float32)),
        scratch_types=[
            pltpu.VMEM((2, GPC, EG), jnp.int32),
            pltpu.VMEM((CH, DQ), jnp.float32),
            pltpu.VMEM_SHARED((npad, DQ), jnp.float32),
            pltpu.SemaphoreType.DMA,
            pltpu.SemaphoreType.DMA,
            pltpu.SemaphoreType.DMA,
            pltpu.SemaphoreType.DMA,
        ],
    )(functools.partial(_deg_body, nchunks, rpn))


def _make_seg_sum(npad, ne_pad):
    nchunks = ne_pad // (NTILES * CH)
    rpn = npad // NTILES
    mesh = plsc.VectorSubcoreMesh(
        core_axis_name="c", subcore_axis_name="s",
        num_cores=2, num_subcores=NTILES)
    return functools.partial(
        pl.kernel,
        mesh=mesh,
        compiler_params=pltpu.CompilerParams(use_tc_tiling_on_sc=False),
        out_type=(jax.ShapeDtypeStruct((npad, DH), jnp.float32),
                  jax.ShapeDtypeStruct((npad, DH), jnp.float32)),
        scratch_types=[
            pltpu.VMEM((NBUF, 2 * GPC, EG), jnp.int32),
            pltpu.VMEM((NBUF, CH, DH), jnp.float32),
            pltpu.VMEM_SHARED((npad, DH), jnp.float32),
        ] + [pltpu.SemaphoreType.DMA] * 9,
    )(functools.partial(_seg_sum_body, nchunks, rpn))


def _prep_tc(emb_blk, ta_blk, tb_blk, dinv_blk, h0_blk, h1_blk):
    deg = ta_blk[:, 0:1] + tb_blk[:, 0:1]
    dinv = jnp.where(deg > 0.0,
                     jax.lax.rsqrt(jnp.maximum(deg, 1e-12)), 0.0)
    dinvh = jnp.broadcast_to(dinv, (deg.shape[0], DH))
    dinv_blk[...] = dinvh
    h0_blk[...] = dinvh * emb_blk[:, :DH]
    h1_blk[...] = dinvh * emb_blk[:, DH:]


def _finalize_tc(t0_blk, t1_blk, dinv_blk, tot_blk, h0_blk, h1_blk, out_blk):
    dinvh = dinv_blk[...]
    g0 = dinvh * t0_blk[...]
    g1 = dinvh * t1_blk[...]
    h0_blk[...] = dinvh * g0
    h1_blk[...] = dinvh * g1
    out_blk[:, :DH] = tot_blk[:, :DH] + g0
    out_blk[:, DH:] = tot_blk[:, DH:] + g1


def _last_tc(t0_blk, t1_blk, dinv_blk, tot_blk, out_blk):
    dinvh = dinv_blk[...]
    out_blk[:, :DH] = (tot_blk[:, :DH] + dinvh * t0_blk[...]) * 0.25
    out_blk[:, DH:] = (tot_blk[:, DH:] + dinvh * t1_blk[...]) * 0.25


def kernel(user_emb, item_emb, user_idx, item_idx):
    n_users = user_emb.shape[0]
    n_items = item_emb.shape[0]
    n_nodes = n_users + n_items
    n_edges = user_idx.shape[0]
    dim = user_emb.shape[1]

    ndummy = 512
    npad = ((n_nodes + ndummy + 127) // 128) * 128
    ne = 2 * n_edges
    qt = NTILES * CH * NBUF
    qd = 4 * NTILES * CH
    lcm = qt * qd // math.gcd(qt, qd)
    ne_pad = ((ne + lcm - 1) // lcm) * lcm
    rpn = npad // NTILES

    src = jnp.concatenate([user_idx, item_idx + n_users])
    dst = jnp.concatenate([item_idx + n_users, user_idx])
    pad = n_nodes + (jnp.arange(ne_pad - ne, dtype=jnp.int32) % ndummy)
    srcp = jnp.concatenate([src, pad])
    dstp = jnp.concatenate([dst, pad])
    dst2 = dstp.reshape(ne_pad // EG, EG)
    tchunks = ne_pad // CH
    comb = jnp.concatenate(
        [srcp.reshape(tchunks, GPC, EG), dstp.reshape(tchunks, GPC, EG)],
        axis=1).reshape(tchunks * 2 * GPC, EG)

    emb = jnp.concatenate([user_emb, item_emb], axis=0)
    emb = jnp.concatenate(
        [emb, jnp.zeros((npad - n_nodes, dim), jnp.float32)], axis=0)
    onesq = jnp.ones((npad, DQ), jnp.float32)
    z16 = jnp.zeros((rpn, DQ), jnp.float32)
    z32 = jnp.zeros((rpn, DH), jnp.float32)

    seg_sum = _make_seg_sum(npad, ne_pad)

    ta, tb = _make_deg(npad, ne_pad)(dst2, onesq, z16)

    nblk = 16
    rb = npad // nblk
    row_spec16 = pl.BlockSpec((rb, DQ), lambda i: (i, 0))
    row_spec32 = pl.BlockSpec((rb, DH), lambda i: (i, 0))
    row_spec64 = pl.BlockSpec((rb, dim), lambda i: (i, 0))
    sdh = jax.ShapeDtypeStruct((npad, DH), jnp.float32)
    dinvh, h0, h1 = pl.pallas_call(
        _prep_tc,
        grid=(nblk,),
        in_specs=[row_spec64, row_spec16, row_spec16],
        out_specs=[row_spec32] * 3,
        out_shape=[sdh] * 3,
    )(emb, ta, tb)

    total = emb
    for layer in range(3):
        t0, t1 = seg_sum(comb, h0, h1, z32)
        if layer < 2:
            h0, h1, total = pl.pallas_call(
                _finalize_tc,
                grid=(nblk,),
                in_specs=[row_spec32] * 3 + [row_spec64],
                out_specs=[row_spec32, row_spec32, row_spec64],
                out_shape=[sdh, sdh,
                           jax.ShapeDtypeStruct((npad, dim), jnp.float32)],
            )(t0, t1, dinvh, total)
        else:
            out = pl.pallas_call(
                _last_tc,
                grid=(nblk,),
                in_specs=[row_spec32] * 3 + [row_spec64],
                out_specs=row_spec64,
                out_shape=jax.ShapeDtypeStruct((npad, dim), jnp.float32),
            )(t0, t1, dinvh, total)

    return out[:n_users], out[n_users:n_nodes]
